# Initial kernel scaffold; baseline (speedup 1.0000x reference)
import jax
import jax.numpy as jnp
from jax.experimental import pallas as pl


def kernel(rgb, depth):
    raise NotImplementedError("write your pallas kernel here")


# speedup vs baseline: 12.5361x; 12.5361x over previous
import jax
import jax.numpy as jnp
from jax import lax
from jax.experimental import pallas as pl

_H = 512
_W = 512
_N = _H * _W
_K_SEL = _N - int(0.9 * _N)


def _body(x_ref, stat_ref, rays_ref):
    x = x_ref[...]
    j = lax.broadcasted_iota(jnp.int32, (_H, 3 * _W), 1)
    ch = j % 3
    wrow = jnp.where(ch == 0, 0.299, jnp.where(ch == 1, 0.587, 0.114)).astype(jnp.float32)
    wx = x * wrow

    sj = lax.broadcasted_iota(jnp.int32, (3 * _W, _W), 0)
    sp = lax.broadcasted_iota(jnp.int32, (3 * _W, _W), 1)
    S = (sj // 3 == sp).astype(jnp.float32)
    lum = lax.dot_general(wx, S, (((1,), (0,)), ((), ())),
                          preferred_element_type=jnp.float32,
                          precision=lax.Precision.HIGHEST)

    lmax = jnp.max(lum)
    lmean = jnp.sum(lum) / _N

    def bis(_, lohi):
        lo, hi = lohi
        mid = 0.5 * (lo + hi)
        c = jnp.sum((lum > mid).astype(jnp.float32))
        big = c >= _K_SEL
        return jnp.where(big, mid, lo), jnp.where(big, hi, mid)

    lo, _ = lax.fori_loop(0, 20, bis, (jnp.float32(0.0), jnp.float32(1.0)))

    selc = (lum > lo).astype(jnp.float32)
    cnt = jnp.sum(selc)
    ej = lax.broadcasted_iota(jnp.int32, (_W, 3 * _W), 1)
    ep = lax.broadcasted_iota(jnp.int32, (_W, 3 * _W), 0)
    E = (ej // 3 == ep).astype(jnp.float32)
    sel_e = lax.dot_general(selc, E, (((1,), (0,)), ((), ())),
                            preferred_element_type=jnp.float32)
    xm = x * sel_e
    rsum = jnp.sum(jnp.where(ch == 0, xm, 0.0))
    gsum = jnp.sum(jnp.where(ch == 1, xm, 0.0))
    bsum = jnp.sum(jnp.where(ch == 2, xm, 0.0))

    ii = lax.broadcasted_iota(jnp.int32, (1, 8), 1)
    statv = jnp.where(ii == 0, rsum / cnt,
            jnp.where(ii == 1, gsum / cnt,
            jnp.where(ii == 2, bsum / cnt,
            jnp.where(ii == 3, lmax,
            jnp.where(ii == 4, lmean, 0.0))))).astype(jnp.float32)
    stat_ref[...] = statv

    yf = lax.broadcasted_iota(jnp.int32, (_H, 3 * _W), 0).astype(jnp.float32)
    xpix = (j // 3).astype(jnp.float32)
    rays_ref[...] = jnp.where(ch == 0, (xpix - 256.0) / 500.0,
                              jnp.where(ch == 1, (yf - 256.0) / 500.0, 1.0))


def kernel(rgb, depth):
    del depth
    x = rgb.reshape(_H, 3 * _W)
    stat, rays = pl.pallas_call(
        _body,
        out_shape=[
            jax.ShapeDtypeStruct((1, 8), jnp.float32),
            jax.ShapeDtypeStruct((_H, 3 * _W), jnp.float32),
        ],
    )(x)
    rgb_mean = stat[0, 0:3][None, :]
    lum = stat[0, 3:5][None, :]
    rays_d = rays.reshape(_H, _W, 3)
    return rgb_mean, lum, rays_d
